# baseline (speedup 1.0000x reference)
import functools

import jax
import jax.numpy as jnp
from jax import lax
from jax.experimental import pallas as pl
from jax.experimental.pallas import tpu as pltpu
from jax.experimental.pallas import tpu_sc as plsc

_BS = 2048

_NC = 2
_NS = 16
_NW = _NC * _NS
_R = 32


def _tc_add_kernel(x_ref, pos_ref, o_ref):
    o_ref[0] = x_ref[0] + pos_ref[...]


def _tc_add(x, pos):
    batch, seq_len, d_model = x.shape
    grid = (seq_len // _BS, batch)
    return pl.pallas_call(
        _tc_add_kernel,
        grid=grid,
        in_specs=[
            pl.BlockSpec((1, _BS, d_model), lambda i, j: (j, i, 0)),
            pl.BlockSpec((_BS, d_model), lambda i, j: (i, 0)),
        ],
        out_specs=pl.BlockSpec((1, _BS, d_model), lambda i, j: (j, i, 0)),
        out_shape=jax.ShapeDtypeStruct(x.shape, x.dtype),
    )(x, pos)


def _sc_add(x, pos):
    batch, seq_len, d_model = x.shape
    spw = seq_len // _NW
    n_chunks = spw // _R
    chunk_w = _R * d_model

    mesh = plsc.VectorSubcoreMesh(
        core_axis_name="c", subcore_axis_name="s",
        num_cores=_NC, num_subcores=_NS)

    @functools.partial(
        pl.kernel,
        out_type=jax.ShapeDtypeStruct((batch * seq_len * d_model,), jnp.float32),
        mesh=mesh,
        scratch_types=[
            pltpu.VMEM((chunk_w,), jnp.float32),
            pltpu.VMEM((chunk_w,), jnp.float32),
        ],
    )
    def sc_kernel(x_hbm, pos_hbm, o_hbm, xbuf, pbuf):
        wid = lax.axis_index("s") * _NC + lax.axis_index("c")

        def chunk(s, _):
            seq0 = wid * spw + s * _R
            pltpu.sync_copy(pos_hbm.at[pl.ds(seq0 * d_model, chunk_w)], pbuf)

            def per_batch(b, _):
                off = (b * seq_len + seq0) * d_model
                pltpu.sync_copy(x_hbm.at[pl.ds(off, chunk_w)], xbuf)

                def add16(i, _):
                    sl = pl.ds(i * 16, 16)
                    xbuf[sl] = xbuf[sl] + pbuf[sl]
                    return 0

                lax.fori_loop(0, chunk_w // 16, add16, 0)
                pltpu.sync_copy(xbuf, o_hbm.at[pl.ds(off, chunk_w)])
                return 0

            lax.fori_loop(0, batch, per_batch, 0)
            return 0

        lax.fori_loop(0, n_chunks, chunk, 0)

    out = sc_kernel(x.reshape(-1), pos.reshape(-1))
    return out.reshape(x.shape)


def kernel(x, pos_table):
    seq_len = x.shape[1]
    pos = pos_table[:seq_len]
    return _sc_add(x, pos)


# speedup vs baseline: 1.7710x; 1.7710x over previous
import functools

import jax
import jax.numpy as jnp
from jax import lax
from jax.experimental import pallas as pl
from jax.experimental.pallas import tpu as pltpu
from jax.experimental.pallas import tpu_sc as plsc

_BS = 2048

_NC = 2
_NS = 16
_NW = _NC * _NS
_R = 16


def _tc_add_kernel(x_ref, pos_ref, o_ref):
    o_ref[0] = x_ref[0] + pos_ref[...]


def _tc_add(x, pos):
    batch, seq_len, d_model = x.shape
    grid = (seq_len // _BS, batch)
    return pl.pallas_call(
        _tc_add_kernel,
        grid=grid,
        in_specs=[
            pl.BlockSpec((1, _BS, d_model), lambda i, j: (j, i, 0)),
            pl.BlockSpec((_BS, d_model), lambda i, j: (i, 0)),
        ],
        out_specs=pl.BlockSpec((1, _BS, d_model), lambda i, j: (j, i, 0)),
        out_shape=jax.ShapeDtypeStruct(x.shape, x.dtype),
    )(x, pos)


def _sc_add(x, pos):
    batch, seq_len, d_model = x.shape
    spw = seq_len // _NW
    n_chunks = spw // _R
    chunk_w = _R * d_model
    n_tasks = n_chunks * batch

    mesh = plsc.VectorSubcoreMesh(
        core_axis_name="c", subcore_axis_name="s",
        num_cores=_NC, num_subcores=_NS)

    @functools.partial(
        pl.kernel,
        out_type=jax.ShapeDtypeStruct((batch * seq_len * d_model,), jnp.float32),
        mesh=mesh,
        scratch_types=(
            [pltpu.VMEM((chunk_w,), jnp.float32)] * 6
            + [pltpu.SemaphoreType.DMA] * 6
        ),
    )
    def sc_kernel(x_hbm, pos_hbm, o_hbm,
                  xb0, xb1, pb0, pb1, ob0, ob1,
                  is0, is1, ps0, ps1, os0, os1):
        wid = lax.axis_index("s") * _NC + lax.axis_index("c")
        seq_base = wid * spw
        xb, pb, ob = [xb0, xb1], [pb0, pb1], [ob0, ob1]
        isem, psem, osem = [is0, is1], [ps0, ps1], [os0, os1]

        def x_slice(t):
            s, b = divmod(t, batch)
            off = (b * seq_len + seq_base + s * _R) * d_model
            return pl.ds(off, chunk_w)

        def p_slice(s):
            return pl.ds((seq_base + s * _R) * d_model, chunk_w)

        pltpu.async_copy(pos_hbm.at[p_slice(0)], pb[0], psem[0])
        if n_chunks > 1:
            pltpu.async_copy(pos_hbm.at[p_slice(1)], pb[1], psem[1])
        pltpu.async_copy(x_hbm.at[x_slice(0)], xb[0], isem[0])
        if n_tasks > 1:
            pltpu.async_copy(x_hbm.at[x_slice(1)], xb[1], isem[1])

        for t in range(n_tasks):
            i = t % 2
            s, b = divmod(t, batch)
            pltpu.make_async_copy(x_hbm.at[x_slice(t)], xb[i], isem[i]).wait()
            if b == 0:
                pltpu.make_async_copy(
                    pos_hbm.at[p_slice(s)], pb[s % 2], psem[s % 2]).wait()
            if t >= 2:
                pltpu.make_async_copy(
                    ob[i], o_hbm.at[x_slice(t - 2)], osem[i]).wait()

            xbi, pbi, obi = xb[i], pb[s % 2], ob[i]

            @plsc.parallel_loop(0, chunk_w, 16, unroll=8)
            def _add(off):
                sl = pl.ds(off, 16)
                obi[sl] = xbi[sl] + pbi[sl]

            if t + 2 < n_tasks:
                pltpu.async_copy(x_hbm.at[x_slice(t + 2)], xb[i], isem[i])
            if b == batch - 1 and s + 2 < n_chunks:
                pltpu.async_copy(
                    pos_hbm.at[p_slice(s + 2)], pb[s % 2], psem[s % 2])
            pltpu.async_copy(ob[i], o_hbm.at[x_slice(t)], osem[i])

        for t in range(max(0, n_tasks - 2), n_tasks):
            i = t % 2
            pltpu.make_async_copy(ob[i], o_hbm.at[x_slice(t)], osem[i]).wait()

    out = sc_kernel(x.reshape(-1), pos.reshape(-1))
    return out.reshape(x.shape)


def kernel(x, pos_table):
    seq_len = x.shape[1]
    pos = pos_table[:seq_len]
    return _sc_add(x, pos)


# speedup vs baseline: 7.5852x; 4.2831x over previous
import functools

import jax
import jax.numpy as jnp
from jax import lax
from jax.experimental import pallas as pl
from jax.experimental.pallas import tpu as pltpu
from jax.experimental.pallas import tpu_sc as plsc

_BS = 2048

_NC = 2
_NS = 16
_NW = _NC * _NS
_R = 16


def _tc_add_kernel(x_ref, pos_ref, o_ref):
    o_ref[0] = x_ref[0] + pos_ref[...]


def _tc_add(x, pos):
    batch, seq_len, d_model = x.shape
    grid = (seq_len // _BS, batch)
    return pl.pallas_call(
        _tc_add_kernel,
        grid=grid,
        in_specs=[
            pl.BlockSpec((1, _BS, d_model), lambda i, j: (j, i, 0)),
            pl.BlockSpec((_BS, d_model), lambda i, j: (i, 0)),
        ],
        out_specs=pl.BlockSpec((1, _BS, d_model), lambda i, j: (j, i, 0)),
        out_shape=jax.ShapeDtypeStruct(x.shape, x.dtype),
    )(x, pos)


def _sc_add(x, pos):
    batch, seq_len, d_model = x.shape
    spw = seq_len // _NW
    n_chunks = spw // _R
    chunk_w = _R * d_model
    n_tasks = n_chunks * batch

    mesh = plsc.VectorSubcoreMesh(
        core_axis_name="c", subcore_axis_name="s",
        num_cores=_NC, num_subcores=_NS)

    @functools.partial(
        pl.kernel,
        out_type=jax.ShapeDtypeStruct((batch * seq_len * d_model,), jnp.float32),
        mesh=mesh,
        scratch_types=(
            [pltpu.VMEM((chunk_w,), jnp.float32)] * 6
            + [pltpu.SemaphoreType.DMA] * 6
        ),
    )
    def sc_kernel(x_hbm, pos_hbm, o_hbm,
                  xb0, xb1, pb0, pb1, ob0, ob1,
                  is0, is1, ps0, ps1, os0, os1):
        wid = lax.axis_index("s") * _NC + lax.axis_index("c")
        seq_base = wid * spw
        xb, pb, ob = [xb0, xb1], [pb0, pb1], [ob0, ob1]
        isem, psem, osem = [is0, is1], [ps0, ps1], [os0, os1]

        def x_slice(t):
            s, b = divmod(t, batch)
            off = (b * seq_len + seq_base + s * _R) * d_model
            return pl.ds(off, chunk_w)

        def p_slice(s):
            return pl.ds((seq_base + s * _R) * d_model, chunk_w)

        pltpu.async_copy(pos_hbm.at[p_slice(0)], pb[0], psem[0])
        if n_chunks > 1:
            pltpu.async_copy(pos_hbm.at[p_slice(1)], pb[1], psem[1])
        pltpu.async_copy(x_hbm.at[x_slice(0)], xb[0], isem[0])
        if n_tasks > 1:
            pltpu.async_copy(x_hbm.at[x_slice(1)], xb[1], isem[1])

        for t in range(n_tasks):
            i = t % 2
            s, b = divmod(t, batch)
            pltpu.make_async_copy(x_hbm.at[x_slice(t)], xb[i], isem[i]).wait()
            if b == 0:
                pltpu.make_async_copy(
                    pos_hbm.at[p_slice(s)], pb[s % 2], psem[s % 2]).wait()
            if t >= 2:
                pltpu.make_async_copy(
                    ob[i], o_hbm.at[x_slice(t - 2)], osem[i]).wait()

            xbi, pbi, obi = xb[i], pb[s % 2], ob[i]

            @plsc.parallel_loop(0, chunk_w, 16, unroll=8)
            def _add(off):
                sl = pl.ds(off, 16)
                obi[sl] = xbi[sl] + pbi[sl]

            if t + 2 < n_tasks:
                pltpu.async_copy(x_hbm.at[x_slice(t + 2)], xb[i], isem[i])
            if b == batch - 1 and s + 2 < n_chunks:
                pltpu.async_copy(
                    pos_hbm.at[p_slice(s + 2)], pb[s % 2], psem[s % 2])
            pltpu.async_copy(ob[i], o_hbm.at[x_slice(t)], osem[i])

        for t in range(max(0, n_tasks - 2), n_tasks):
            i = t % 2
            pltpu.make_async_copy(ob[i], o_hbm.at[x_slice(t)], osem[i]).wait()

    out = sc_kernel(x.reshape(-1), pos.reshape(-1))
    return out.reshape(x.shape)


def kernel(x, pos_table):
    seq_len = x.shape[1]
    pos = pos_table[:seq_len]
    return _tc_add(x, pos)
